# Initial kernel scaffold; baseline (speedup 1.0000x reference)
import jax
import jax.numpy as jnp
from jax.experimental import pallas as pl


def kernel(x, rbf, idx_i, idx_j, Wk, Wi, bi, Wj, bj, Wri1, bri1, Wri2, bri2, Wd, bd, u, Wra1, bra1, Wra2, bra2):
    raise NotImplementedError("write your pallas kernel here")


# speedup vs baseline: 2.7539x; 2.7539x over previous
import functools

import jax
import jax.numpy as jnp
from jax import lax
from jax.experimental import pallas as pl
from jax.experimental.pallas import tpu as pltpu
from jax.experimental.pallas import tpu_sc as plsc

NC = 2
NS = 16
NW = NC * NS
LANES = 16


def _edge_filters(rbf, Wk):
    E, K = rbf.shape
    Fo = Wk.shape[1]
    BE = 3200
    assert E % BE == 0

    def body(rbf_ref, wk_ref, out_ref):
        out_ref[...] = jnp.dot(rbf_ref[...], wk_ref[...],
                               preferred_element_type=jnp.float32)

    return pl.pallas_call(
        body,
        grid=(E // BE,),
        in_specs=[
            pl.BlockSpec((BE, K), lambda i: (i, 0)),
            pl.BlockSpec((K, Fo), lambda i: (0, 0)),
        ],
        out_specs=pl.BlockSpec((BE, Fo), lambda i: (i, 0)),
        out_shape=jax.ShapeDtypeStruct((E, Fo), jnp.float32),
    )(rbf, Wk)


def _project(x, W, b):
    N, F = x.shape
    BN = 2000
    assert N % BN == 0

    def body(x_ref, w_ref, b_ref, out_ref):
        out_ref[...] = (
            jnp.dot(x_ref[...], w_ref[...], preferred_element_type=jnp.float32)
            + b_ref[...]
        )

    return pl.pallas_call(
        body,
        grid=(N // BN,),
        in_specs=[
            pl.BlockSpec((BN, F), lambda i: (i, 0)),
            pl.BlockSpec((F, F), lambda i: (0, 0)),
            pl.BlockSpec((1, F), lambda i: (0, 0)),
        ],
        out_specs=pl.BlockSpec((BN, F), lambda i: (i, 0)),
        out_shape=jax.ShapeDtypeStruct((N, F), jnp.float32),
    )(x, W, b.reshape(1, F))


def _sc_messages(g, xj_src, idx_i, idx_j):
    E, F = g.shape
    N = xj_src.shape[0]
    EPW = E // NW
    C = 80
    assert EPW % C == 0
    steps = EPW // C
    NCHUNK = N // C
    assert N % C == 0
    CPT = -(-NCHUNK // NS)

    mesh = plsc.VectorSubcoreMesh(core_axis_name="c", subcore_axis_name="s")

    @functools.partial(
        pl.kernel,
        mesh=mesh,
        out_type=jax.ShapeDtypeStruct((NC, N, F), jnp.float32),
        scratch_types=[
            pltpu.VMEM((C,), jnp.int32),
            pltpu.VMEM((C,), jnp.int32),
            pltpu.VMEM((C, F), jnp.float32),
            pltpu.VMEM((C, F), jnp.float32),
            pltpu.VMEM_SHARED((N, F), jnp.float32),
            pltpu.SemaphoreType.DMA,
        ],
    )
    def k(g_hbm, xjsrc_hbm, idxi_hbm, idxj_hbm, out_hbm,
          idxj_v, idxi_v, g_v, rows_v, acc, sem):
        c = lax.axis_index("c")
        s = lax.axis_index("s")
        wid = c * NS + s

        def _zrow(r, carry):
            for j in range(F // LANES):
                g_v[r, pl.ds(j * LANES, LANES)] = jnp.zeros((LANES,), jnp.float32)
            return carry
        lax.fori_loop(0, C, _zrow, 0)
        for t in range(CPT):
            q = s + t * NS

            @pl.when(q < NCHUNK)
            def _():
                pltpu.sync_copy(g_v, acc.at[pl.ds(q * C, C)])
        plsc.subcore_barrier()

        base0 = wid * EPW

        def _step(i, carry):
            base = base0 + i * C
            pltpu.sync_copy(idxj_hbm.at[pl.ds(base, C)], idxj_v)
            pltpu.sync_copy(idxi_hbm.at[pl.ds(base, C)], idxi_v)
            gather = pltpu.async_copy(xjsrc_hbm.at[idxj_v], rows_v, sem)
            pltpu.sync_copy(g_hbm.at[pl.ds(base, C)], g_v)
            gather.wait()

            def _mrow(r, inner):
                for j in range(F // LANES):
                    sl = pl.ds(j * LANES, LANES)
                    g_v[r, sl] = g_v[r, sl] * rows_v[r, sl]
                return inner
            lax.fori_loop(0, C, _mrow, 0)

            pltpu.sync_copy(g_v, acc.at[idxi_v], add=True)
            return carry
        lax.fori_loop(0, steps, _step, 0)

        plsc.subcore_barrier()

        for t in range(CPT):
            q = s + t * NS

            @pl.when(q < NCHUNK)
            def _():
                pltpu.sync_copy(acc.at[pl.ds(q * C, C)], g_v)
                pltpu.sync_copy(g_v, out_hbm.at[c, pl.ds(q * C, C)])

    return k(g, xj_src, idx_i, idx_j)


def _tail(x, parts, Wi, bi, Wri1, bri1, Wri2, bri2, Wd, bd, u, Wra1, bra1,
          Wra2, bra2):
    N, F = x.shape
    LR = Wri1.shape[0]
    BN = 2000
    assert N % BN == 0

    def body(x_ref, p_ref, wi, bi_, wri1, bri1_, wri2, bri2_, wd, bd_, u_,
             wra1, bra1_, wra2, bra2_, out_ref):
        xb = x_ref[...]
        p = p_ref[...]
        m = (jnp.dot(xb, wi[...], preferred_element_type=jnp.float32)
             + bi_[...] + p[0] + p[1])
        for l in range(LR):
            t = (jnp.dot(m, wri1[l], preferred_element_type=jnp.float32)
                 + bri1_[...][l].reshape(1, F))
            m = m + (jnp.dot(t, wri2[l], preferred_element_type=jnp.float32)
                     + bri2_[...][l].reshape(1, F))
        x1 = u_[...] * xb + (jnp.dot(m, wd[...], preferred_element_type=jnp.float32)
                             + bd_[...])
        for l in range(LR):
            t = (jnp.dot(x1, wra1[l], preferred_element_type=jnp.float32)
                 + bra1_[...][l].reshape(1, F))
            x1 = x1 + (jnp.dot(t, wra2[l], preferred_element_type=jnp.float32)
                       + bra2_[...][l].reshape(1, F))
        out_ref[...] = x1

    full = lambda shape: pl.BlockSpec(shape, lambda i: tuple(0 for _ in shape))
    return pl.pallas_call(
        body,
        grid=(N // BN,),
        in_specs=[
            pl.BlockSpec((BN, F), lambda i: (i, 0)),
            pl.BlockSpec((NC, BN, F), lambda i: (0, i, 0)),
            full((F, F)),
            full((1, F)),
            full((LR, F, F)),
            full((LR, F)),
            full((LR, F, F)),
            full((LR, F)),
            full((F, F)),
            full((1, F)),
            full((1, F)),
            full((LR, F, F)),
            full((LR, F)),
            full((LR, F, F)),
            full((LR, F)),
        ],
        out_specs=pl.BlockSpec((BN, F), lambda i: (i, 0)),
        out_shape=jax.ShapeDtypeStruct((N, F), jnp.float32),
    )(x, parts, Wi, bi.reshape(1, F), Wri1, bri1, Wri2, bri2, Wd,
      bd.reshape(1, F), u.reshape(1, F), Wra1, bra1, Wra2, bra2)


def kernel(x, rbf, idx_i, idx_j, Wk, Wi, bi, Wj, bj, Wri1, bri1, Wri2, bri2,
           Wd, bd, u, Wra1, bra1, Wra2, bra2):
    g = _edge_filters(rbf, Wk)
    xj_src = _project(x, Wj, bj)
    parts = _sc_messages(g, xj_src, idx_i, idx_j)
    return _tail(x, parts, Wi, bi, Wri1, bri1, Wri2, bri2, Wd, bd, u,
                 Wra1, bra1, Wra2, bra2)
